# baseline (speedup 1.0000x reference)
import functools

import jax
import jax.numpy as jnp
from jax import lax
from jax.experimental import pallas as pl
from jax.experimental.pallas import tpu as pltpu
from jax.experimental.pallas import tpu_sc as plsc

K = 64
N = 128
NWORD = N // 32
LLR_MAX = 100.0
TAU = 3e-6
NC, NS, L = 2, 16, 16
NW = NC * NS
BS = 512
EPW = BS // NW


def _worker_id():
    return lax.axis_index("s") * NC + lax.axis_index("c")


def _sc_body(llr_hbm, gml_hbm, out_hbm, llr_v, a_v, st_v, lv_v, d_v, v_v, o_v):
    w = _worker_id()
    lane = lax.broadcasted_iota(jnp.int32, (L,), 0)

    pltpu.sync_copy(llr_hbm.at[w], llr_v)
    pltpu.sync_copy(gml_hbm, st_v)

    def prep(j, _):
        x = jnp.clip(llr_v[pl.ds(j * L, L)], -LLR_MAX, LLR_MAX)
        llr_v[pl.ds(j * L, L)] = x
        a_v[pl.ds(j * L, L)] = jnp.abs(x)
        return 0

    lax.fori_loop(0, N, prep, 0, unroll=8)

    def step(i, _):
        i4 = i * NWORD
        rws = [st_v[pl.ds((i4 + t) * L, L)] for t in range(NWORD)]
        t2s = list(rws)
        bests = [jnp.full((L,), -1.0, jnp.float32) for _ in range(NWORD)]
        jsels = [jnp.zeros((L,), jnp.int32) for _ in range(NWORD)]
        for b in range(31, -1, -1):
            for t in range(NWORD):
                j = t * 32 + b
                aj = a_v[j * L:(j + 1) * L]
                m = (t2s[t] < 0) & (aj >= bests[t])
                bests[t] = jnp.where(m, aj, bests[t])
                jsels[t] = jnp.where(m, j, jsels[t])
                t2s[t] = lax.shift_left(t2s[t], 1)
        best, jsel = bests[0], jsels[0]
        for t in range(1, NWORD):
            m = bests[t] > best
            best = jnp.where(m, bests[t], best)
            jsel = jnp.where(m, jsels[t], jsel)
        lv_v[pl.ds(i * L, L)] = plsc.load_gather(llr_v, [jsel * L + lane])
        jw = lax.shift_right_logical(jsel, 5)
        jb2 = 31 - (jsel & 31)
        mjw = [jw == t for t in range(1, NWORD)]

        def rowupd(r, _):
            k = r * NWORD * L
            sw = [st_v[pl.ds(k + t * L, L)] for t in range(NWORD)]
            tw = sw[0]
            for t in range(1, NWORD):
                tw = jnp.where(mjw[t - 1], sw[t], tw)
            msk = lax.shift_right_arithmetic(lax.shift_left(tw, jb2), 31)
            for t in range(NWORD):
                st_v[pl.ds(k + t * L, L)] = sw[t] ^ (msk & rws[t])
            return 0

        lax.fori_loop(0, K, rowupd, 0, unroll=8)
        for t in range(NWORD):
            st_v[pl.ds((i4 + t) * L, L)] = rws[t]
        return 0

    lax.fori_loop(0, K, step, 0)

    def cacc(i, cw):
        u = (lv_v[pl.ds(i * L, L)] > 0.0).astype(jnp.int32)
        m = -u
        return tuple(cw[t] ^ (m & st_v[pl.ds((i * NWORD + t) * L, L)])
                     for t in range(NWORD))

    cws = lax.fori_loop(0, K, cacc,
                        tuple(jnp.zeros((L,), jnp.int32) for _ in range(NWORD)),
                        unroll=8)

    for t in range(NWORD):
        tw = cws[t]
        for b in range(32):
            j = t * 32 + b
            cb = (tw & 1).astype(jnp.float32)
            x = llr_v[j * L:(j + 1) * L]
            v_v[j * L:(j + 1) * L] = x - 2.0 * cb * x
            tw = lax.shift_right_logical(tw, 1)

    zero = jnp.zeros((L,), jnp.float32)

    def drow(i, dm):
        i4 = i * NWORD
        accs = []
        for t in range(NWORD):
            t2 = st_v[pl.ds((i4 + t) * L, L)]
            acc = zero
            for b in range(31, -1, -1):
                j = t * 32 + b
                acc = acc + jnp.where(t2 < 0, v_v[j * L:(j + 1) * L], 0.0)
                t2 = lax.shift_left(t2, 1)
            accs.append(acc)
        d = (accs[0] + accs[1]) + (accs[2] + accs[3])
        d_v[pl.ds(i * L, L)] = d
        return jnp.maximum(dm, d)

    dmax = lax.fori_loop(0, K, drow, jnp.full((L,), -jnp.inf, jnp.float32))

    def firstsel(i, isel):
        hit = (isel >= K) & (d_v[pl.ds(i * L, L)] >= dmax - TAU)
        return jnp.where(hit, i, isel)

    isel = lax.fori_loop(0, K, firstsel, jnp.full((L,), K, jnp.int32),
                         unroll=8)
    dsel = plsc.load_gather(d_v, [isel * L + lane])
    fm = -(dsel > TAU).astype(jnp.int32)

    ews = [plsc.load_gather(st_v, [(isel * NWORD + t) * L + lane]) & fm
           for t in range(NWORD)]
    for t in range(NWORD):
        ow = cws[t] ^ ews[t]
        for b in range(32):
            j = t * 32 + b
            o_v[j * L:(j + 1) * L] = (ow & 1).astype(jnp.float32)
            ow = lax.shift_right_logical(ow, 1)

    pltpu.sync_copy(o_v, out_hbm.at[w])


def _make_sc_kernel(interpret=False):
    return functools.partial(
        pl.kernel,
        out_type=jax.ShapeDtypeStruct((NW, N * EPW), jnp.float32),
        mesh=plsc.VectorSubcoreMesh(core_axis_name="c", subcore_axis_name="s",
                                    num_cores=NC, num_subcores=NS),
        scratch_types=[
            pltpu.VMEM((N * L,), jnp.float32),
            pltpu.VMEM((N * L,), jnp.float32),
            pltpu.VMEM((K * NWORD * L,), jnp.int32),
            pltpu.VMEM((K * L,), jnp.float32),
            pltpu.VMEM((K * L,), jnp.float32),
            pltpu.VMEM((N * L,), jnp.float32),
            pltpu.VMEM((N * L,), jnp.float32),
        ],
        compiler_params=pltpu.CompilerParams(needs_layout_passes=False),
        interpret=interpret,
    )(_sc_body)


@jax.jit
def kernel(inputs, gm):
    shape = inputs.shape
    llr = inputs.reshape(-1, N).astype(jnp.float32)
    bs = llr.shape[0]
    gmi = gm.astype(jnp.int32)
    shifts = jnp.arange(32, dtype=jnp.int32)
    gmb = (gmi.reshape(K, NWORD, 32) << shifts[None, None, :]).sum(
        axis=-1, dtype=jnp.int32)
    gml = jnp.broadcast_to(gmb.reshape(K * NWORD, 1), (K * NWORD, L))
    gml = jnp.asarray(gml, jnp.int32).reshape(K * NWORD * L)
    llr3 = llr.reshape(NW, EPW, N).transpose(0, 2, 1)
    out3 = _make_sc_kernel()(llr3.reshape(NW, N * EPW), gml)
    out = out3.reshape(NW, N, EPW).transpose(0, 2, 1).reshape(bs, N)
    return out.reshape(shape)


# speedup vs baseline: 1.1114x; 1.1114x over previous
import functools

import jax
import jax.numpy as jnp
from jax import lax
from jax.experimental import pallas as pl
from jax.experimental.pallas import tpu as pltpu
from jax.experimental.pallas import tpu_sc as plsc

K = 64
N = 128
NWORD = N // 32
LLR_MAX = 100.0
TAU = 3e-6
NC, NS, L = 2, 16, 16
NW = NC * NS
BS = 512
EPW = BS // NW


def _worker_id():
    return lax.axis_index("s") * NC + lax.axis_index("c")


def _sc_body(llr_hbm, gml_hbm, out_hbm, llr_v, a_v, st_v, lv_v, d_v, v_v, o_v):
    w = _worker_id()
    lane = lax.broadcasted_iota(jnp.int32, (L,), 0)

    pltpu.sync_copy(llr_hbm.at[w], llr_v)
    pltpu.sync_copy(gml_hbm, st_v)

    def prep(j, _):
        x = jnp.clip(llr_v[pl.ds(j * L, L)], -LLR_MAX, LLR_MAX)
        llr_v[pl.ds(j * L, L)] = x
        a_v[pl.ds(j * L, L)] = jnp.abs(x)
        return 0

    lax.fori_loop(0, N, prep, 0, unroll=8)

    def step(i, _):
        i4 = i * NWORD
        rws = [st_v[pl.ds((i4 + t) * L, L)] for t in range(NWORD)]
        t2s = list(rws)
        bests = [jnp.full((L,), -1.0, jnp.float32) for _ in range(NWORD)]
        jsels = [jnp.zeros((L,), jnp.int32) for _ in range(NWORD)]
        for b in range(31, -1, -1):
            for t in range(NWORD):
                j = t * 32 + b
                aj = a_v[j * L:(j + 1) * L]
                m = (t2s[t] < 0) & (aj >= bests[t])
                bests[t] = jnp.where(m, aj, bests[t])
                jsels[t] = jnp.where(m, j, jsels[t])
                t2s[t] = lax.shift_left(t2s[t], 1)
        best, jsel = bests[0], jsels[0]
        for t in range(1, NWORD):
            m = bests[t] > best
            best = jnp.where(m, bests[t], best)
            jsel = jnp.where(m, jsels[t], jsel)
        lv_v[pl.ds(i * L, L)] = plsc.load_gather(llr_v, [jsel * L + lane])
        jw = lax.shift_right_logical(jsel, 5)
        jb2 = 31 - (jsel & 31)
        mjw = [jw == t for t in range(1, NWORD)]

        def rowupd(r, _):
            k = r * NWORD * L
            sw = [st_v[pl.ds(k + t * L, L)] for t in range(NWORD)]
            tw = sw[0]
            for t in range(1, NWORD):
                tw = jnp.where(mjw[t - 1], sw[t], tw)
            msk = lax.shift_right_arithmetic(lax.shift_left(tw, jb2), 31)
            for t in range(NWORD):
                st_v[pl.ds(k + t * L, L)] = sw[t] ^ (msk & rws[t])
            return 0

        lax.fori_loop(0, K, rowupd, 0, unroll=8)
        for t in range(NWORD):
            st_v[pl.ds((i4 + t) * L, L)] = rws[t]
        return 0

    lax.fori_loop(0, K, step, 0)

    def cacc(i, cw):
        u = (lv_v[pl.ds(i * L, L)] > 0.0).astype(jnp.int32)
        m = -u
        return tuple(cw[t] ^ (m & st_v[pl.ds((i * NWORD + t) * L, L)])
                     for t in range(NWORD))

    cws = lax.fori_loop(0, K, cacc,
                        tuple(jnp.zeros((L,), jnp.int32) for _ in range(NWORD)))

    for t in range(NWORD):
        tw = cws[t]
        for b in range(32):
            j = t * 32 + b
            cb = (tw & 1).astype(jnp.float32)
            x = llr_v[j * L:(j + 1) * L]
            v_v[j * L:(j + 1) * L] = x - 2.0 * cb * x
            tw = lax.shift_right_logical(tw, 1)

    zero = jnp.zeros((L,), jnp.float32)

    def drow(i, _):
        i4 = i * NWORD
        accs = []
        for t in range(NWORD):
            t2 = st_v[pl.ds((i4 + t) * L, L)]
            acc = zero
            for b in range(31, -1, -1):
                j = t * 32 + b
                acc = acc + jnp.where(t2 < 0, v_v[j * L:(j + 1) * L], 0.0)
                t2 = lax.shift_left(t2, 1)
            accs.append(acc)
        d_v[pl.ds(i * L, L)] = (accs[0] + accs[1]) + (accs[2] + accs[3])
        return 0

    lax.fori_loop(0, K, drow, 0)

    def dmaxf(i, dm):
        return jnp.maximum(dm, d_v[pl.ds(i * L, L)])

    dmax = lax.fori_loop(0, K, dmaxf, jnp.full((L,), -jnp.inf, jnp.float32))

    def firstsel(i, isel):
        hit = (isel >= K) & (d_v[pl.ds(i * L, L)] >= dmax - TAU)
        return jnp.where(hit, i, isel)

    isel = lax.fori_loop(0, K, firstsel, jnp.full((L,), K, jnp.int32))
    dsel = plsc.load_gather(d_v, [isel * L + lane])
    fm = -(dsel > TAU).astype(jnp.int32)

    ews = [plsc.load_gather(st_v, [(isel * NWORD + t) * L + lane]) & fm
           for t in range(NWORD)]
    for t in range(NWORD):
        ow = cws[t] ^ ews[t]
        for b in range(32):
            j = t * 32 + b
            o_v[j * L:(j + 1) * L] = (ow & 1).astype(jnp.float32)
            ow = lax.shift_right_logical(ow, 1)

    pltpu.sync_copy(o_v, out_hbm.at[w])


def _make_sc_kernel(interpret=False):
    return functools.partial(
        pl.kernel,
        out_type=jax.ShapeDtypeStruct((NW, N * EPW), jnp.float32),
        mesh=plsc.VectorSubcoreMesh(core_axis_name="c", subcore_axis_name="s",
                                    num_cores=NC, num_subcores=NS),
        scratch_types=[
            pltpu.VMEM((N * L,), jnp.float32),
            pltpu.VMEM((N * L,), jnp.float32),
            pltpu.VMEM((K * NWORD * L,), jnp.int32),
            pltpu.VMEM((K * L,), jnp.float32),
            pltpu.VMEM((K * L,), jnp.float32),
            pltpu.VMEM((N * L,), jnp.float32),
            pltpu.VMEM((N * L,), jnp.float32),
        ],
        compiler_params=pltpu.CompilerParams(needs_layout_passes=False),
        interpret=interpret,
    )(_sc_body)


@jax.jit
def kernel(inputs, gm):
    shape = inputs.shape
    llr = inputs.reshape(-1, N).astype(jnp.float32)
    bs = llr.shape[0]
    gmi = gm.astype(jnp.int32)
    shifts = jnp.arange(32, dtype=jnp.int32)
    gmb = (gmi.reshape(K, NWORD, 32) << shifts[None, None, :]).sum(
        axis=-1, dtype=jnp.int32)
    gml = jnp.broadcast_to(gmb.reshape(K * NWORD, 1), (K * NWORD, L))
    gml = jnp.asarray(gml, jnp.int32).reshape(K * NWORD * L)
    llr3 = llr.reshape(NW, EPW, N).transpose(0, 2, 1)
    out3 = _make_sc_kernel()(llr3.reshape(NW, N * EPW), gml)
    out = out3.reshape(NW, N, EPW).transpose(0, 2, 1).reshape(bs, N)
    return out.reshape(shape)
